# Initial kernel scaffold; baseline (speedup 1.0000x reference)
import jax
import jax.numpy as jnp
from jax.experimental import pallas as pl


def kernel(x, edge_index, W1, b1, W2, b2):
    raise NotImplementedError("write your pallas kernel here")


# speedup vs baseline: 14.9033x; 14.9033x over previous
import functools

import jax
import jax.numpy as jnp
from jax import lax
from jax.experimental import pallas as pl
from jax.experimental.pallas import tpu as pltpu
from jax.experimental.pallas import tpu_sc as plsc

N_NODES = 10000
N_EDGES = 320000
D_FEAT = 128
HIDDEN = 128
N_CLASSES = 16

NC = 2
NS = 16
NW = NC * NS

NP = 10240
RPT = NP // NS
EPW = N_EDGES // NW
K = 80
NCHUNK = EPW // K

_mesh = plsc.VectorSubcoreMesh(core_axis_name="c", subcore_axis_name="s")


def _zero_vmem_2d(ref, rows, cols):
    z16 = jnp.zeros((16,), jnp.float32)

    def body(i, carry):
        for j in range(cols // 16):
            ref[i, pl.ds(j * 16, 16)] = z16
        return carry

    lax.fori_loop(0, rows, body, 0)


def _zero_vmem_1d(ref, n):
    z16 = jnp.zeros((16,), jnp.float32)

    def body(i, carry):
        ref[pl.ds(i * 16, 16)] = z16
        return carry

    lax.fori_loop(0, n // 16, body, 0)


@functools.partial(
    pl.kernel,
    out_type=jax.ShapeDtypeStruct((NC, NP), jnp.float32),
    mesh=_mesh,
    scratch_types=[
        pltpu.VMEM((K,), jnp.int32),
        pltpu.VMEM((K,), jnp.float32),
        pltpu.VMEM((RPT,), jnp.float32),
        pltpu.VMEM_SHARED((NP,), jnp.float32),
    ],
)
def _deg_kernel(dst_hbm, out_hbm, dst_buf, ones_buf, zstage, acc):
    c = lax.axis_index("c")
    s = lax.axis_index("s")
    wid = c * NS + s
    ebase = wid * EPW
    r0 = s * RPT

    one16 = jnp.ones((16,), jnp.float32)
    for j in range(K // 16):
        ones_buf[pl.ds(j * 16, 16)] = one16
    _zero_vmem_1d(zstage, RPT)
    pltpu.sync_copy(zstage, acc.at[pl.ds(r0, RPT)])
    plsc.subcore_barrier()

    def body(g, carry):
        eb = pl.multiple_of(ebase + g * K, 8)
        pltpu.sync_copy(dst_hbm.at[pl.ds(eb, K)], dst_buf)
        pltpu.sync_copy(ones_buf, acc.at[dst_buf], add=True)
        return carry

    lax.fori_loop(0, NCHUNK, body, 0)
    plsc.subcore_barrier()
    pltpu.sync_copy(acc.at[pl.ds(r0, RPT)], out_hbm.at[c, pl.ds(r0, RPT)])


def _make_agg_kernel(d):
    @functools.partial(
        pl.kernel,
        out_type=jax.ShapeDtypeStruct((NC, NP, d), jnp.float32),
        mesh=_mesh,
        compiler_params=pltpu.CompilerParams(
            use_tc_tiling_on_sc=False) if d < 128 else None,
        scratch_types=[
            pltpu.VMEM((K,), jnp.int32),
            pltpu.VMEM((K,), jnp.int32),
            pltpu.VMEM((K, d), jnp.float32),
            pltpu.VMEM((64, d), jnp.float32),
            pltpu.VMEM_SHARED((NP, d), jnp.float32),
        ],
    )
    def agg(src_hbm, dst_hbm, u_hbm, out_hbm, src_buf, dst_buf, rows_buf,
            zbuf, acc):
        c = lax.axis_index("c")
        s = lax.axis_index("s")
        wid = c * NS + s
        ebase = wid * EPW
        r0 = s * RPT

        _zero_vmem_2d(zbuf, 64, d)
        for j in range(RPT // 64):
            pltpu.sync_copy(zbuf, acc.at[pl.ds(r0 + j * 64, 64), :])
        plsc.subcore_barrier()

        def body(g, carry):
            eb = pl.multiple_of(ebase + g * K, 8)
            pltpu.sync_copy(src_hbm.at[pl.ds(eb, K)], src_buf)
            pltpu.sync_copy(dst_hbm.at[pl.ds(eb, K)], dst_buf)
            pltpu.sync_copy(u_hbm.at[src_buf], rows_buf)
            pltpu.sync_copy(rows_buf, acc.at[dst_buf], add=True)
            return carry

        lax.fori_loop(0, NCHUNK, body, 0)
        plsc.subcore_barrier()
        pltpu.sync_copy(acc.at[pl.ds(r0, RPT), :],
                        out_hbm.at[c, pl.ds(r0, RPT), :])

    return agg


_agg128 = _make_agg_kernel(D_FEAT)
_agg16 = _make_agg_kernel(N_CLASSES)


_R = 1024


def _u1_body(x_ref, w_ref, d0_ref, d1_ref, u_ref, dis_ref):
    deg = d0_ref[...] + d1_ref[...] + 1.0
    dis = lax.rsqrt(deg)
    h = jnp.dot(x_ref[...], w_ref[...], preferred_element_type=jnp.float32)
    u_ref[...] = h * dis
    dis_ref[...] = dis


def _u1_call(x, w1, deg0, deg1):
    grid = (NP // _R,)
    return pl.pallas_call(
        _u1_body,
        grid=grid,
        in_specs=[
            pl.BlockSpec((_R, D_FEAT), lambda i: (i, 0)),
            pl.BlockSpec((D_FEAT, HIDDEN), lambda i: (0, 0)),
            pl.BlockSpec((_R, 1), lambda i: (i, 0)),
            pl.BlockSpec((_R, 1), lambda i: (i, 0)),
        ],
        out_specs=[
            pl.BlockSpec((_R, HIDDEN), lambda i: (i, 0)),
            pl.BlockSpec((_R, 1), lambda i: (i, 0)),
        ],
        out_shape=[
            jax.ShapeDtypeStruct((NP, HIDDEN), jnp.float32),
            jax.ShapeDtypeStruct((NP, 1), jnp.float32),
        ],
    )(x, w1, deg0, deg1)


def _mid_body(s1_ref, u1_ref, dis_ref, b1_ref, w2_ref, u2_ref):
    agg = (s1_ref[0] + s1_ref[1] + u1_ref[...]) * dis_ref[...]
    z = jnp.maximum(agg + b1_ref[...], 0.0)
    h2 = jnp.dot(z, w2_ref[...], preferred_element_type=jnp.float32)
    u2_ref[...] = h2 * dis_ref[...]


def _mid_call(s1, u1, dis, b1, w2):
    grid = (NP // _R,)
    return pl.pallas_call(
        _mid_body,
        grid=grid,
        in_specs=[
            pl.BlockSpec((NC, _R, HIDDEN), lambda i: (0, i, 0)),
            pl.BlockSpec((_R, HIDDEN), lambda i: (i, 0)),
            pl.BlockSpec((_R, 1), lambda i: (i, 0)),
            pl.BlockSpec((1, HIDDEN), lambda i: (0, 0)),
            pl.BlockSpec((HIDDEN, N_CLASSES), lambda i: (0, 0)),
        ],
        out_specs=pl.BlockSpec((_R, N_CLASSES), lambda i: (i, 0)),
        out_shape=jax.ShapeDtypeStruct((NP, N_CLASSES), jnp.float32),
    )(s1, u1, dis, b1, w2)


def _out_body(s2_ref, u2_ref, dis_ref, b2_ref, o_ref):
    logit = (s2_ref[0] + s2_ref[1] + u2_ref[...]) * dis_ref[...] + b2_ref[...]
    m = jnp.max(logit, axis=1, keepdims=True)
    e = jnp.exp(logit - m)
    lse = jnp.log(jnp.sum(e, axis=1, keepdims=True)) + m
    o_ref[...] = logit - lse


def _out_call(s2, u2, dis, b2):
    grid = (NP // _R,)
    return pl.pallas_call(
        _out_body,
        grid=grid,
        in_specs=[
            pl.BlockSpec((NC, _R, N_CLASSES), lambda i: (0, i, 0)),
            pl.BlockSpec((_R, N_CLASSES), lambda i: (i, 0)),
            pl.BlockSpec((_R, 1), lambda i: (i, 0)),
            pl.BlockSpec((1, N_CLASSES), lambda i: (0, 0)),
        ],
        out_specs=pl.BlockSpec((_R, N_CLASSES), lambda i: (i, 0)),
        out_shape=jax.ShapeDtypeStruct((NP, N_CLASSES), jnp.float32),
    )(s2, u2, dis, b2)


@jax.jit
def kernel(x, edge_index, W1, b1, W2, b2):
    src = edge_index[0].astype(jnp.int32)
    dst = edge_index[1].astype(jnp.int32)

    x_pad = jnp.zeros((NP, D_FEAT), jnp.float32).at[:N_NODES].set(x)

    degp = _deg_kernel(dst)
    deg0 = degp[0][:, None]
    deg1 = degp[1][:, None]

    u1, dis = _u1_call(x_pad, W1, deg0, deg1)
    s1 = _agg128(src, dst, u1)
    u2 = _mid_call(s1, u1, dis, b1[None, :], W2)
    s2 = _agg16(src, dst, u2)
    out = _out_call(s2, u2, dis, b2[None, :])
    return out[:N_NODES]


# speedup vs baseline: 27.9353x; 1.8744x over previous
import functools

import jax
import jax.numpy as jnp
from jax import lax
from jax.experimental import pallas as pl
from jax.experimental.pallas import tpu as pltpu
from jax.experimental.pallas import tpu_sc as plsc

N_NODES = 10000
N_EDGES = 320000
D_FEAT = 128
HIDDEN = 128
N_CLASSES = 16

NC = 2
NS = 16
NW = NC * NS

NP = 10240
RPT = NP // NS
EPW = N_EDGES // NW
K = 80
NCHUNK = EPW // K

_mesh = plsc.VectorSubcoreMesh(core_axis_name="c", subcore_axis_name="s")


def _copy_row(src2d, row, dst1d, n):
    for j in range(n // 16):
        dst1d[pl.ds(j * 16, 16)] = src2d[row, pl.ds(j * 16, 16)]


def _zero_vmem_2d(ref, rows, cols):
    z16 = jnp.zeros((16,), jnp.float32)

    def body(i, carry):
        for j in range(cols // 16):
            ref[i, pl.ds(j * 16, 16)] = z16
        return carry

    lax.fori_loop(0, rows, body, 0)


@functools.partial(
    pl.kernel,
    out_type=jax.ShapeDtypeStruct((NC, NP), jnp.float32),
    mesh=_mesh,
    compiler_params=pltpu.CompilerParams(use_tc_tiling_on_sc=False),
    scratch_types=[
        pltpu.VMEM((NCHUNK, K), jnp.int32),
        pltpu.VMEM((K,), jnp.int32),
        pltpu.VMEM((K,), jnp.float32),
        pltpu.VMEM((RPT,), jnp.float32),
        pltpu.VMEM_SHARED((NP,), jnp.float32),
    ],
)
def _deg_kernel(dst_hbm, out_hbm, idxd, db, ones_buf, zstage, acc):
    c = lax.axis_index("c")
    s = lax.axis_index("s")
    wid = c * NS + s
    r0 = s * RPT

    one16 = jnp.ones((16,), jnp.float32)
    z16 = jnp.zeros((16,), jnp.float32)
    for j in range(K // 16):
        ones_buf[pl.ds(j * 16, 16)] = one16

    def zb(i, carry):
        zstage[pl.ds(i * 16, 16)] = z16
        return carry

    lax.fori_loop(0, RPT // 16, zb, 0)
    pltpu.sync_copy(dst_hbm.at[wid], idxd)
    pltpu.sync_copy(zstage, acc.at[pl.ds(r0, RPT)])
    plsc.subcore_barrier()

    def body(g, carry):
        _copy_row(idxd, g, db, K)
        pltpu.sync_copy(ones_buf, acc.at[db], add=True)
        return carry

    lax.fori_loop(0, NCHUNK, body, 0)
    plsc.subcore_barrier()
    pltpu.sync_copy(acc.at[pl.ds(r0, RPT)], out_hbm.at[c, pl.ds(r0, RPT)])


def _make_agg_kernel(d):
    @functools.partial(
        pl.kernel,
        out_type=jax.ShapeDtypeStruct((NC, NP, d), jnp.float32),
        mesh=_mesh,
        compiler_params=pltpu.CompilerParams(use_tc_tiling_on_sc=False),
        scratch_types=[
            pltpu.VMEM((NCHUNK, K), jnp.int32),
            pltpu.VMEM((NCHUNK, K), jnp.int32),
            pltpu.VMEM((K,), jnp.int32),
            pltpu.VMEM((K,), jnp.int32),
            pltpu.VMEM((K,), jnp.int32),
            pltpu.VMEM((K,), jnp.int32),
            pltpu.VMEM((K, d), jnp.float32),
            pltpu.VMEM((K, d), jnp.float32),
            pltpu.VMEM((16, d), jnp.float32),
            pltpu.VMEM_SHARED((NP, d), jnp.float32),
            pltpu.SemaphoreType.DMA,
            pltpu.SemaphoreType.DMA,
        ],
    )
    def agg(src_hbm, dst_hbm, u_hbm, out_hbm, idxs, idxd, sb0, sb1, db0, db1,
            rows0, rows1, zbuf, acc, sem0, sem1):
        c = lax.axis_index("c")
        s = lax.axis_index("s")
        wid = c * NS + s
        r0 = s * RPT

        pltpu.sync_copy(src_hbm.at[wid], idxs)
        pltpu.sync_copy(dst_hbm.at[wid], idxd)
        _zero_vmem_2d(zbuf, 16, d)

        def zinit(j, carry):
            pltpu.sync_copy(zbuf, acc.at[pl.ds(r0 + j * 16, 16), :])
            return carry

        lax.fori_loop(0, RPT // 16, zinit, 0)
        plsc.subcore_barrier()

        _copy_row(idxs, 0, sb0, K)
        pltpu.async_copy(u_hbm.at[sb0], rows0, sem0)

        def body(g, carry):
            c0 = 2 * g
            _copy_row(idxs, c0 + 1, sb1, K)
            _copy_row(idxd, c0, db0, K)
            pltpu.make_async_copy(u_hbm.at[sb0], rows0, sem0).wait()
            pltpu.async_copy(u_hbm.at[sb1], rows1, sem1)
            pltpu.sync_copy(rows0, acc.at[db0], add=True)
            _copy_row(idxs, c0 + 2, sb0, K)
            _copy_row(idxd, c0 + 1, db1, K)
            pltpu.make_async_copy(u_hbm.at[sb1], rows1, sem1).wait()
            pltpu.async_copy(u_hbm.at[sb0], rows0, sem0)
            pltpu.sync_copy(rows1, acc.at[db1], add=True)
            return carry

        lax.fori_loop(0, (NCHUNK - 1) // 2, body, 0)
        _copy_row(idxd, NCHUNK - 1, db0, K)
        pltpu.make_async_copy(u_hbm.at[sb0], rows0, sem0).wait()
        pltpu.sync_copy(rows0, acc.at[db0], add=True)

        plsc.subcore_barrier()
        pltpu.sync_copy(acc.at[pl.ds(r0, RPT), :],
                        out_hbm.at[c, pl.ds(r0, RPT), :])

    return agg


_agg128 = _make_agg_kernel(D_FEAT)
_agg16 = _make_agg_kernel(N_CLASSES)


_R = 1024


def _u1_body(x_ref, w_ref, d0_ref, d1_ref, u_ref, dis_ref):
    deg = d0_ref[...] + d1_ref[...] + 1.0
    dis = lax.rsqrt(deg)
    h = jnp.dot(x_ref[...], w_ref[...], preferred_element_type=jnp.float32)
    u_ref[...] = h * dis
    dis_ref[...] = dis


def _u1_call(x, w1, deg0, deg1):
    grid = (NP // _R,)
    return pl.pallas_call(
        _u1_body,
        grid=grid,
        in_specs=[
            pl.BlockSpec((_R, D_FEAT), lambda i: (i, 0)),
            pl.BlockSpec((D_FEAT, HIDDEN), lambda i: (0, 0)),
            pl.BlockSpec((_R, 1), lambda i: (i, 0)),
            pl.BlockSpec((_R, 1), lambda i: (i, 0)),
        ],
        out_specs=[
            pl.BlockSpec((_R, HIDDEN), lambda i: (i, 0)),
            pl.BlockSpec((_R, 1), lambda i: (i, 0)),
        ],
        out_shape=[
            jax.ShapeDtypeStruct((NP, HIDDEN), jnp.float32),
            jax.ShapeDtypeStruct((NP, 1), jnp.float32),
        ],
    )(x, w1, deg0, deg1)


def _mid_body(s1_ref, u1_ref, dis_ref, b1_ref, w2_ref, u2_ref):
    agg = (s1_ref[0] + s1_ref[1] + u1_ref[...]) * dis_ref[...]
    z = jnp.maximum(agg + b1_ref[...], 0.0)
    h2 = jnp.dot(z, w2_ref[...], preferred_element_type=jnp.float32)
    u2_ref[...] = h2 * dis_ref[...]


def _mid_call(s1, u1, dis, b1, w2):
    grid = (NP // _R,)
    return pl.pallas_call(
        _mid_body,
        grid=grid,
        in_specs=[
            pl.BlockSpec((NC, _R, HIDDEN), lambda i: (0, i, 0)),
            pl.BlockSpec((_R, HIDDEN), lambda i: (i, 0)),
            pl.BlockSpec((_R, 1), lambda i: (i, 0)),
            pl.BlockSpec((1, HIDDEN), lambda i: (0, 0)),
            pl.BlockSpec((HIDDEN, N_CLASSES), lambda i: (0, 0)),
        ],
        out_specs=pl.BlockSpec((_R, N_CLASSES), lambda i: (i, 0)),
        out_shape=jax.ShapeDtypeStruct((NP, N_CLASSES), jnp.float32),
    )(s1, u1, dis, b1, w2)


def _out_body(s2_ref, u2_ref, dis_ref, b2_ref, o_ref):
    logit = (s2_ref[0] + s2_ref[1] + u2_ref[...]) * dis_ref[...] + b2_ref[...]
    m = jnp.max(logit, axis=1, keepdims=True)
    e = jnp.exp(logit - m)
    lse = jnp.log(jnp.sum(e, axis=1, keepdims=True)) + m
    o_ref[...] = logit - lse


def _out_call(s2, u2, dis, b2):
    grid = (NP // _R,)
    return pl.pallas_call(
        _out_body,
        grid=grid,
        in_specs=[
            pl.BlockSpec((NC, _R, N_CLASSES), lambda i: (0, i, 0)),
            pl.BlockSpec((_R, N_CLASSES), lambda i: (i, 0)),
            pl.BlockSpec((_R, 1), lambda i: (i, 0)),
            pl.BlockSpec((1, N_CLASSES), lambda i: (0, 0)),
        ],
        out_specs=pl.BlockSpec((_R, N_CLASSES), lambda i: (i, 0)),
        out_shape=jax.ShapeDtypeStruct((NP, N_CLASSES), jnp.float32),
    )(s2, u2, dis, b2)


@jax.jit
def kernel(x, edge_index, W1, b1, W2, b2):
    src = edge_index[0].astype(jnp.int32).reshape(NW, NCHUNK, K)
    dst = edge_index[1].astype(jnp.int32).reshape(NW, NCHUNK, K)

    x_pad = jnp.zeros((NP, D_FEAT), jnp.float32).at[:N_NODES].set(x)

    degp = _deg_kernel(dst)
    deg0 = degp[0][:, None]
    deg1 = degp[1][:, None]

    u1, dis = _u1_call(x_pad, W1, deg0, deg1)
    s1 = _agg128(src, dst, u1)
    u2 = _mid_call(s1, u1, dis, b1[None, :], W2)
    s2 = _agg16(src, dst, u2)
    out = _out_call(s2, u2, dis, b2[None, :])
    return out[:N_NODES]


# speedup vs baseline: 32.4936x; 1.1632x over previous
import functools

import jax
import jax.numpy as jnp
from jax import lax
from jax.experimental import pallas as pl
from jax.experimental.pallas import tpu as pltpu
from jax.experimental.pallas import tpu_sc as plsc

N_NODES = 10000
N_EDGES = 320000
D_FEAT = 128
HIDDEN = 128
N_CLASSES = 16

NC = 2
NS = 16
NW = NC * NS

NP = 10240
RPT = NP // NS
EPW = N_EDGES // NW
K = 80
NCHUNK = EPW // K

_mesh = plsc.VectorSubcoreMesh(core_axis_name="c", subcore_axis_name="s")


def _copy_row(src2d, row, dst1d, n):
    for j in range(n // 16):
        dst1d[pl.ds(j * 16, 16)] = src2d[row, pl.ds(j * 16, 16)]


def _zero_vmem_2d(ref, rows, cols):
    z16 = jnp.zeros((16,), jnp.float32)

    def body(i, carry):
        for j in range(cols // 16):
            ref[i, pl.ds(j * 16, 16)] = z16
        return carry

    lax.fori_loop(0, rows, body, 0)


@functools.partial(
    pl.kernel,
    out_type=jax.ShapeDtypeStruct((NC, NP), jnp.float32),
    mesh=_mesh,
    compiler_params=pltpu.CompilerParams(use_tc_tiling_on_sc=False),
    scratch_types=[
        pltpu.VMEM((NCHUNK, K), jnp.int32),
        pltpu.VMEM((K,), jnp.int32),
        pltpu.VMEM((K,), jnp.float32),
        pltpu.VMEM((RPT,), jnp.float32),
        pltpu.VMEM_SHARED((NP,), jnp.float32),
    ],
)
def _deg_kernel(dst_hbm, out_hbm, idxd, db, ones_buf, zstage, acc):
    c = lax.axis_index("c")
    s = lax.axis_index("s")
    wid = c * NS + s
    r0 = s * RPT

    one16 = jnp.ones((16,), jnp.float32)
    z16 = jnp.zeros((16,), jnp.float32)
    for j in range(K // 16):
        ones_buf[pl.ds(j * 16, 16)] = one16

    def zb(i, carry):
        zstage[pl.ds(i * 16, 16)] = z16
        return carry

    lax.fori_loop(0, RPT // 16, zb, 0)
    pltpu.sync_copy(dst_hbm.at[wid], idxd)
    pltpu.sync_copy(zstage, acc.at[pl.ds(r0, RPT)])
    plsc.subcore_barrier()

    def body(g, carry):
        _copy_row(idxd, g, db, K)
        pltpu.sync_copy(ones_buf, acc.at[db], add=True)
        return carry

    lax.fori_loop(0, NCHUNK, body, 0)
    plsc.subcore_barrier()
    pltpu.sync_copy(acc.at[pl.ds(r0, RPT)], out_hbm.at[c, pl.ds(r0, RPT)])


def _make_agg_kernel(d, nbuf):
    niter = (NCHUNK - nbuf) // nbuf
    rem = NCHUNK - niter * nbuf - nbuf

    scratch = [
        pltpu.VMEM((NCHUNK, K), jnp.int32),
        pltpu.VMEM((NCHUNK, K), jnp.int32),
    ]
    scratch += [pltpu.VMEM((K,), jnp.int32) for _ in range(nbuf)]
    scratch += [pltpu.VMEM((K,), jnp.int32) for _ in range(nbuf)]
    scratch += [pltpu.VMEM((K, d), jnp.float32) for _ in range(nbuf)]
    scratch += [
        pltpu.VMEM((8, d), jnp.float32),
        pltpu.VMEM_SHARED((NP, d), jnp.float32),
    ]
    scratch += [pltpu.SemaphoreType.DMA for _ in range(2 * nbuf)]

    @functools.partial(
        pl.kernel,
        out_type=jax.ShapeDtypeStruct((NC, NP, d), jnp.float32),
        mesh=_mesh,
        compiler_params=pltpu.CompilerParams(use_tc_tiling_on_sc=False),
        scratch_types=scratch,
    )
    def agg(src_hbm, dst_hbm, u_hbm, out_hbm, idxs, idxd, *rest):
        sb = rest[:nbuf]
        db = rest[nbuf:2 * nbuf]
        rows = rest[2 * nbuf:3 * nbuf]
        zbuf = rest[3 * nbuf]
        acc = rest[3 * nbuf + 1]
        semg = rest[3 * nbuf + 2:3 * nbuf + 2 + nbuf]
        sems = rest[3 * nbuf + 2 + nbuf:]

        c = lax.axis_index("c")
        s = lax.axis_index("s")
        wid = c * NS + s
        r0 = s * RPT

        pltpu.sync_copy(src_hbm.at[wid], idxs)
        pltpu.sync_copy(dst_hbm.at[wid], idxd)
        _zero_vmem_2d(zbuf, 8, d)

        def zinit(j, carry):
            pltpu.sync_copy(zbuf, acc.at[pl.ds(r0 + j * 8, 8), :])
            return carry

        lax.fori_loop(0, RPT // 8, zinit, 0)
        plsc.subcore_barrier()

        def gather(b, chunk):
            pltpu.async_copy(u_hbm.at[sb[b]], rows[b], semg[b])

        def wait_gather(b):
            pltpu.make_async_copy(u_hbm.at[sb[b]], rows[b], semg[b]).wait()

        def scatter(b):
            pltpu.async_copy(rows[b], acc.at[db[b]], sems[b], add=True)

        def wait_scatter(b):
            pltpu.make_async_copy(rows[b], acc.at[db[b]], sems[b]).wait()

        for b in range(nbuf):
            _copy_row(idxs, b, sb[b], K)
            _copy_row(idxd, b, db[b], K)
            gather(b, b)

        def body(g, carry):
            c0 = nbuf * g
            for b in range(nbuf):
                wait_gather(b)
                scatter(b)
            for b in range(nbuf):
                wait_scatter(b)
                _copy_row(idxs, c0 + nbuf + b, sb[b], K)
                _copy_row(idxd, c0 + nbuf + b, db[b], K)
                gather(b, c0 + nbuf + b)
            return carry

        lax.fori_loop(0, niter, body, 0)

        for b in range(nbuf):
            wait_gather(b)
            scatter(b)
        for b in range(nbuf):
            wait_scatter(b)
        for t in range(rem):
            cc = niter * nbuf + nbuf + t
            _copy_row(idxs, cc, sb[0], K)
            _copy_row(idxd, cc, db[0], K)
            pltpu.sync_copy(u_hbm.at[sb[0]], rows[0])
            pltpu.sync_copy(rows[0], acc.at[db[0]], add=True)

        plsc.subcore_barrier()
        pltpu.sync_copy(acc.at[pl.ds(r0, RPT), :],
                        out_hbm.at[c, pl.ds(r0, RPT), :])

    return agg


_agg128 = _make_agg_kernel(D_FEAT, 2)
_agg16 = _make_agg_kernel(N_CLASSES, 4)


_R = 1024


def _u1_body(x_ref, w_ref, d0_ref, d1_ref, u_ref, dis_ref):
    deg = d0_ref[...] + d1_ref[...] + 1.0
    dis = lax.rsqrt(deg)
    h = jnp.dot(x_ref[...], w_ref[...], preferred_element_type=jnp.float32)
    u_ref[...] = h * dis
    dis_ref[...] = dis


def _u1_call(x, w1, deg0, deg1):
    grid = (NP // _R,)
    return pl.pallas_call(
        _u1_body,
        grid=grid,
        in_specs=[
            pl.BlockSpec((_R, D_FEAT), lambda i: (i, 0)),
            pl.BlockSpec((D_FEAT, HIDDEN), lambda i: (0, 0)),
            pl.BlockSpec((_R, 1), lambda i: (i, 0)),
            pl.BlockSpec((_R, 1), lambda i: (i, 0)),
        ],
        out_specs=[
            pl.BlockSpec((_R, HIDDEN), lambda i: (i, 0)),
            pl.BlockSpec((_R, 1), lambda i: (i, 0)),
        ],
        out_shape=[
            jax.ShapeDtypeStruct((NP, HIDDEN), jnp.float32),
            jax.ShapeDtypeStruct((NP, 1), jnp.float32),
        ],
    )(x, w1, deg0, deg1)


def _mid_body(s1_ref, u1_ref, dis_ref, b1_ref, w2_ref, u2_ref):
    agg = (s1_ref[0] + s1_ref[1] + u1_ref[...]) * dis_ref[...]
    z = jnp.maximum(agg + b1_ref[...], 0.0)
    h2 = jnp.dot(z, w2_ref[...], preferred_element_type=jnp.float32)
    u2_ref[...] = h2 * dis_ref[...]


def _mid_call(s1, u1, dis, b1, w2):
    grid = (NP // _R,)
    return pl.pallas_call(
        _mid_body,
        grid=grid,
        in_specs=[
            pl.BlockSpec((NC, _R, HIDDEN), lambda i: (0, i, 0)),
            pl.BlockSpec((_R, HIDDEN), lambda i: (i, 0)),
            pl.BlockSpec((_R, 1), lambda i: (i, 0)),
            pl.BlockSpec((1, HIDDEN), lambda i: (0, 0)),
            pl.BlockSpec((HIDDEN, N_CLASSES), lambda i: (0, 0)),
        ],
        out_specs=pl.BlockSpec((_R, N_CLASSES), lambda i: (i, 0)),
        out_shape=jax.ShapeDtypeStruct((NP, N_CLASSES), jnp.float32),
    )(s1, u1, dis, b1, w2)


def _out_body(s2_ref, u2_ref, dis_ref, b2_ref, o_ref):
    logit = (s2_ref[0] + s2_ref[1] + u2_ref[...]) * dis_ref[...] + b2_ref[...]
    m = jnp.max(logit, axis=1, keepdims=True)
    e = jnp.exp(logit - m)
    lse = jnp.log(jnp.sum(e, axis=1, keepdims=True)) + m
    o_ref[...] = logit - lse


def _out_call(s2, u2, dis, b2):
    grid = (NP // _R,)
    return pl.pallas_call(
        _out_body,
        grid=grid,
        in_specs=[
            pl.BlockSpec((NC, _R, N_CLASSES), lambda i: (0, i, 0)),
            pl.BlockSpec((_R, N_CLASSES), lambda i: (i, 0)),
            pl.BlockSpec((_R, 1), lambda i: (i, 0)),
            pl.BlockSpec((1, N_CLASSES), lambda i: (0, 0)),
        ],
        out_specs=pl.BlockSpec((_R, N_CLASSES), lambda i: (i, 0)),
        out_shape=jax.ShapeDtypeStruct((NP, N_CLASSES), jnp.float32),
    )(s2, u2, dis, b2)


@jax.jit
def kernel(x, edge_index, W1, b1, W2, b2):
    src = edge_index[0].astype(jnp.int32).reshape(NW, NCHUNK, K)
    dst = edge_index[1].astype(jnp.int32).reshape(NW, NCHUNK, K)

    x_pad = jnp.zeros((NP, D_FEAT), jnp.float32).at[:N_NODES].set(x)

    degp = _deg_kernel(dst)
    deg0 = degp[0][:, None]
    deg1 = degp[1][:, None]

    u1, dis = _u1_call(x_pad, W1, deg0, deg1)
    s1 = _agg128(src, dst, u1)
    u2 = _mid_call(s1, u1, dis, b1[None, :], W2)
    s2 = _agg16(src, dst, u2)
    out = _out_call(s2, u2, dis, b2[None, :])
    return out[:N_NODES]
